# baseline (speedup 1.0000x reference)
import functools

import jax
import jax.numpy as jnp
from jax.experimental import pallas as pl
from jax.experimental.pallas import tpu as pltpu

_IN_CHANNELS = [512, 1024, 512, 256, 256, 256]
_NUM_ANCHORS = [4, 6, 6, 6, 4, 4]
_NUM_CLASSES = 91
_FEAT_HW = [64, 32, 16, 8, 4, 2]
_LEVEL_GROUPS = ((0,), (1, 2, 3, 4, 5))


def _head_kernel(*refs, levels):
    nl = len(levels)
    x_refs = refs[0:nl]
    w_refs = refs[nl:2 * nl]
    b_refs = refs[2 * nl:3 * nl]
    cls_refs = refs[3 * nl:4 * nl]
    reg_refs = refs[4 * nl:5 * nl]
    xp_refs = refs[5 * nl:6 * nl]

    first_step = pl.program_id(0) == 0
    for i, lvl in enumerate(levels):
        H = W = _FEAT_HW[lvl]
        A = _NUM_ANCHORS[lvl]
        C = _IN_CHANNELS[lvl]
        HW = H * W
        P = W + 1
        L = HW + 2 * W + 2
        ncls = _NUM_CLASSES * A
        x_ref, w_ref, b_ref = x_refs[i], w_refs[i], b_refs[i]
        cls_ref, reg_ref, xp_ref = cls_refs[i], reg_refs[i], xp_refs[i]

        @pl.when(first_step)
        def _zero_borders(xp_ref=xp_ref, P=P, HW=HW, L=L, C=C):
            xp_ref[pl.ds(0, P), :] = jnp.zeros((P, C), jnp.bfloat16)
            xp_ref[pl.ds(P + HW, L - P - HW), :] = jnp.zeros(
                (L - P - HW, C), jnp.bfloat16)

        xp_ref[pl.ds(P, HW), :] = jnp.transpose(
            x_ref[0].astype(jnp.bfloat16), (1, 0))

        total = None
        for dx in (-1, 0, 1):
            acc = None
            for dy in (-1, 0, 1):
                t = (dy + 1) * 3 + (dx + 1)
                xs = xp_ref[pl.ds(P + dy * W + dx, HW), :]
                m = jnp.dot(xs, w_ref[t], preferred_element_type=jnp.float32)
                acc = m if acc is None else acc + m
            if dx != 0:
                col = jax.lax.broadcasted_iota(jnp.int32, (HW, 1), 0) % W
                bad = col == (0 if dx == -1 else W - 1)
                acc = jnp.where(bad, 0.0, acc)
            total = acc if total is None else total + acc
        total = total + b_ref[...]
        cls_ref[0] = total[:, :ncls]
        reg_ref[0] = total[:, ncls:]


def _group_call(levels, xrs, wcs, bcs, N):
    x_specs, w_specs, b_specs = [], [], []
    cls_specs, reg_specs, out_shapes, scratch_shapes = [], [], [], []
    for lvl in levels:
        C = _IN_CHANNELS[lvl]
        A = _NUM_ANCHORS[lvl]
        W = _FEAT_HW[lvl]
        HW = W * W
        L = HW + 2 * W + 2
        Cout = (_NUM_CLASSES + 4) * A
        x_specs.append(pl.BlockSpec((1, C, HW), lambda n: (n, 0, 0)))
        w_specs.append(pl.BlockSpec((9, C, Cout), lambda n: (0, 0, 0)))
        b_specs.append(pl.BlockSpec((1, Cout), lambda n: (0, 0)))
        cls_specs.append(pl.BlockSpec((1, HW, _NUM_CLASSES * A),
                                      lambda n: (n, 0, 0)))
        reg_specs.append(pl.BlockSpec((1, HW, 4 * A), lambda n: (n, 0, 0)))
        scratch_shapes.append(pltpu.VMEM((L, C), jnp.bfloat16))
    out_shapes = (
        [jax.ShapeDtypeStruct((N, _FEAT_HW[l] ** 2, _NUM_CLASSES * _NUM_ANCHORS[l]),
                              jnp.float32) for l in levels]
        + [jax.ShapeDtypeStruct((N, _FEAT_HW[l] ** 2, 4 * _NUM_ANCHORS[l]),
                                jnp.float32) for l in levels]
    )
    return pl.pallas_call(
        functools.partial(_head_kernel, levels=levels),
        grid=(N,),
        in_specs=x_specs + w_specs + b_specs,
        out_specs=cls_specs + reg_specs,
        out_shape=out_shapes,
        scratch_shapes=scratch_shapes,
        compiler_params=pltpu.CompilerParams(
            dimension_semantics=("arbitrary",),
        ),
    )(*[xrs[l] for l in levels], *[wcs[l] for l in levels],
      *[bcs[l] for l in levels])


def kernel(x0, x1, x2, x3, x4, x5, reg_w0, reg_w1, reg_w2, reg_w3, reg_w4, reg_w5, reg_b0, reg_b1, reg_b2, reg_b3, reg_b4, reg_b5, cls_w0, cls_w1, cls_w2, cls_w3, cls_w4, cls_w5, cls_b0, cls_b1, cls_b2, cls_b3, cls_b4, cls_b5):
    xs = [x0, x1, x2, x3, x4, x5]
    reg_ws = [reg_w0, reg_w1, reg_w2, reg_w3, reg_w4, reg_w5]
    reg_bs = [reg_b0, reg_b1, reg_b2, reg_b3, reg_b4, reg_b5]
    cls_ws = [cls_w0, cls_w1, cls_w2, cls_w3, cls_w4, cls_w5]
    cls_bs = [cls_b0, cls_b1, cls_b2, cls_b3, cls_b4, cls_b5]
    N = x0.shape[0]

    xrs, wcs, bcs = [], [], []
    for i in range(6):
        C = _IN_CHANNELS[i]
        HW = _FEAT_HW[i] ** 2
        Cout = (_NUM_CLASSES + 4) * _NUM_ANCHORS[i]
        xrs.append(xs[i].reshape(N, C, HW))
        wc = jnp.concatenate([cls_ws[i], reg_ws[i]], axis=0)
        wc = jnp.transpose(wc, (2, 3, 1, 0)).reshape(9, C, Cout)
        wcs.append(wc.astype(jnp.bfloat16))
        bcs.append(jnp.concatenate([cls_bs[i], reg_bs[i]])[None, :])

    cls_parts = [None] * 6
    reg_parts = [None] * 6
    for levels in _LEVEL_GROUPS:
        outs = _group_call(levels, xrs, wcs, bcs, N)
        nl = len(levels)
        for j, lvl in enumerate(levels):
            HWA = _FEAT_HW[lvl] ** 2 * _NUM_ANCHORS[lvl]
            cls_parts[lvl] = outs[j].reshape(N, HWA, _NUM_CLASSES)
            reg_parts[lvl] = outs[nl + j].reshape(N, HWA, 4)

    bbox_regression = jnp.concatenate(reg_parts, axis=1)
    cls_logits = jnp.concatenate(cls_parts, axis=1)
    return (bbox_regression, cls_logits)


# speedup vs baseline: 1.0003x; 1.0003x over previous
import functools

import jax
import jax.numpy as jnp
from jax.experimental import pallas as pl
from jax.experimental.pallas import tpu as pltpu

_IN_CHANNELS = [512, 1024, 512, 256, 256, 256]
_NUM_ANCHORS = [4, 6, 6, 6, 4, 4]
_NUM_CLASSES = 91
_FEAT_HW = [64, 32, 16, 8, 4, 2]
_LEVEL_GROUPS = ((0,), (1, 2, 3, 4, 5))


def _head_kernel(*refs, levels):
    nl = len(levels)
    x_refs = refs[0:nl]
    w_refs = refs[nl:2 * nl]
    b_refs = refs[2 * nl:3 * nl]
    cls_refs = refs[3 * nl:4 * nl]
    reg_refs = refs[4 * nl:5 * nl]
    xp_refs = refs[5 * nl:6 * nl]

    first_step = pl.program_id(0) == 0
    for i, lvl in enumerate(levels):
        H = W = _FEAT_HW[lvl]
        A = _NUM_ANCHORS[lvl]
        C = _IN_CHANNELS[lvl]
        HW = H * W
        P = W + 1
        L = HW + 2 * W + 2
        ncls = _NUM_CLASSES * A
        x_ref, w_ref, b_ref = x_refs[i], w_refs[i], b_refs[i]
        cls_ref, reg_ref, xp_ref = cls_refs[i], reg_refs[i], xp_refs[i]

        @pl.when(first_step)
        def _zero_borders(xp_ref=xp_ref, P=P, HW=HW, L=L, C=C):
            xp_ref[pl.ds(0, P), :] = jnp.zeros((P, C), jnp.bfloat16)
            xp_ref[pl.ds(P + HW, L - P - HW), :] = jnp.zeros(
                (L - P - HW, C), jnp.bfloat16)

        xp_ref[pl.ds(P, HW), :] = jnp.transpose(
            x_ref[0].astype(jnp.bfloat16), (1, 0))

        total = None
        for dx in (-1, 0, 1):
            acc = None
            for dy in (-1, 0, 1):
                t = (dy + 1) * 3 + (dx + 1)
                xs = xp_ref[pl.ds(P + dy * W + dx, HW), :]
                m = jnp.dot(xs, w_ref[t], preferred_element_type=jnp.float32)
                acc = m if acc is None else acc + m
            if dx != 0:
                col = jax.lax.broadcasted_iota(jnp.int32, (HW, 1), 0) % W
                bad = col == (0 if dx == -1 else W - 1)
                acc = jnp.where(bad, 0.0, acc)
            total = acc if total is None else total + acc
        total = total + b_ref[...]
        cls_ref[0] = total[:, :ncls]
        reg_ref[0] = total[:, ncls:]


def _group_call(levels, xrs, wcs, bcs, N):
    x_specs, w_specs, b_specs = [], [], []
    cls_specs, reg_specs, out_shapes, scratch_shapes = [], [], [], []
    for lvl in levels:
        C = _IN_CHANNELS[lvl]
        A = _NUM_ANCHORS[lvl]
        W = _FEAT_HW[lvl]
        HW = W * W
        L = HW + 2 * W + 2
        Cout = (_NUM_CLASSES + 4) * A
        x_specs.append(pl.BlockSpec((1, C, HW), lambda n: (n, 0, 0)))
        w_specs.append(pl.BlockSpec((9, C, Cout), lambda n: (0, 0, 0)))
        b_specs.append(pl.BlockSpec((1, Cout), lambda n: (0, 0)))
        cls_specs.append(pl.BlockSpec((1, HW, _NUM_CLASSES * A),
                                      lambda n: (n, 0, 0)))
        reg_specs.append(pl.BlockSpec((1, HW, 4 * A), lambda n: (n, 0, 0)))
        scratch_shapes.append(pltpu.VMEM((L, C), jnp.bfloat16))
    out_shapes = (
        [jax.ShapeDtypeStruct((N, _FEAT_HW[l] ** 2, _NUM_CLASSES * _NUM_ANCHORS[l]),
                              jnp.float32) for l in levels]
        + [jax.ShapeDtypeStruct((N, _FEAT_HW[l] ** 2, 4 * _NUM_ANCHORS[l]),
                                jnp.float32) for l in levels]
    )
    return pl.pallas_call(
        functools.partial(_head_kernel, levels=levels),
        grid=(N,),
        in_specs=x_specs + w_specs + b_specs,
        out_specs=cls_specs + reg_specs,
        out_shape=out_shapes,
        scratch_shapes=scratch_shapes,
        compiler_params=pltpu.CompilerParams(
            dimension_semantics=("arbitrary",),
        ),
    )(*[xrs[l] for l in levels], *[wcs[l] for l in levels],
      *[bcs[l] for l in levels])


def kernel(x0, x1, x2, x3, x4, x5, reg_w0, reg_w1, reg_w2, reg_w3, reg_w4, reg_w5, reg_b0, reg_b1, reg_b2, reg_b3, reg_b4, reg_b5, cls_w0, cls_w1, cls_w2, cls_w3, cls_w4, cls_w5, cls_b0, cls_b1, cls_b2, cls_b3, cls_b4, cls_b5):
    xs = [x0, x1, x2, x3, x4, x5]
    reg_ws = [reg_w0, reg_w1, reg_w2, reg_w3, reg_w4, reg_w5]
    reg_bs = [reg_b0, reg_b1, reg_b2, reg_b3, reg_b4, reg_b5]
    cls_ws = [cls_w0, cls_w1, cls_w2, cls_w3, cls_w4, cls_w5]
    cls_bs = [cls_b0, cls_b1, cls_b2, cls_b3, cls_b4, cls_b5]
    N = x0.shape[0]

    xrs, wcs, bcs = [], [], []
    for i in range(6):
        C = _IN_CHANNELS[i]
        HW = _FEAT_HW[i] ** 2
        Cout = (_NUM_CLASSES + 4) * _NUM_ANCHORS[i]
        xrs.append(xs[i].reshape(N, C, HW))
        wc = jnp.concatenate([cls_ws[i], reg_ws[i]], axis=0)
        wc = wc.astype(jnp.bfloat16)
        wcs.append(jnp.transpose(wc, (2, 3, 1, 0)).reshape(9, C, Cout))
        bcs.append(jnp.concatenate([cls_bs[i], reg_bs[i]])[None, :])

    cls_parts = [None] * 6
    reg_parts = [None] * 6
    for levels in _LEVEL_GROUPS:
        outs = _group_call(levels, xrs, wcs, bcs, N)
        nl = len(levels)
        for j, lvl in enumerate(levels):
            HWA = _FEAT_HW[lvl] ** 2 * _NUM_ANCHORS[lvl]
            cls_parts[lvl] = outs[j].reshape(N, HWA, _NUM_CLASSES)
            reg_parts[lvl] = outs[nl + j].reshape(N, HWA, 4)

    bbox_regression = jnp.concatenate(reg_parts, axis=1)
    cls_logits = jnp.concatenate(cls_parts, axis=1)
    return (bbox_regression, cls_logits)


# speedup vs baseline: 1.1178x; 1.1174x over previous
import functools

import jax
import jax.numpy as jnp
from jax.experimental import pallas as pl
from jax.experimental.pallas import tpu as pltpu

_IN_CHANNELS = [512, 1024, 512, 256, 256, 256]
_NUM_ANCHORS = [4, 6, 6, 6, 4, 4]
_NUM_CLASSES = 91
_FEAT_HW = [64, 32, 16, 8, 4, 2]
_LEVEL_GROUPS = ((0,), (1, 2, 3, 4, 5))


def _head_kernel(*refs, levels):
    nl = len(levels)
    x_refs = refs[0:nl]
    w_refs = refs[nl:2 * nl]
    b_refs = refs[2 * nl:3 * nl]
    cls_refs = refs[3 * nl:4 * nl]
    reg_refs = refs[4 * nl:5 * nl]
    xp_refs = refs[5 * nl:6 * nl]

    first_step = pl.program_id(0) == 0
    for i, lvl in enumerate(levels):
        H = W = _FEAT_HW[lvl]
        A = _NUM_ANCHORS[lvl]
        C = _IN_CHANNELS[lvl]
        HW = H * W
        P = W + 1
        L = HW + 2 * W + 2
        ncls = _NUM_CLASSES * A
        x_ref, w_ref, b_ref = x_refs[i], w_refs[i], b_refs[i]
        cls_ref, reg_ref, xp_ref = cls_refs[i], reg_refs[i], xp_refs[i]

        @pl.when(first_step)
        def _zero_borders(xp_ref=xp_ref, P=P, HW=HW, L=L, C=C):
            xp_ref[pl.ds(0, P), :] = jnp.zeros((P, C), jnp.bfloat16)
            xp_ref[pl.ds(P + HW, L - P - HW), :] = jnp.zeros(
                (L - P - HW, C), jnp.bfloat16)

        xp_ref[pl.ds(P, HW), :] = jnp.transpose(
            x_ref[0].astype(jnp.bfloat16), (1, 0))

        total = None
        for dx in (-1, 0, 1):
            acc = None
            for dy in (-1, 0, 1):
                t = (dy + 1) * 3 + (dx + 1)
                xs = xp_ref[pl.ds(P + dy * W + dx, HW), :]
                m = jnp.dot(xs, w_ref[t], preferred_element_type=jnp.float32)
                acc = m if acc is None else acc + m
            if dx != 0:
                col = jax.lax.broadcasted_iota(jnp.int32, (HW, 1), 0) % W
                bad = col == (0 if dx == -1 else W - 1)
                acc = jnp.where(bad, 0.0, acc)
            total = acc if total is None else total + acc
        total = total + b_ref[...]
        for a in range(A):
            cls_ref[0, pl.Slice(a, HW, A), :] = (
                total[:, a * _NUM_CLASSES:(a + 1) * _NUM_CLASSES])
        reg_ref[0] = total[:, ncls:]


def _group_call(levels, xrs, wcs, bcs, N):
    x_specs, w_specs, b_specs = [], [], []
    cls_specs, reg_specs, out_shapes, scratch_shapes = [], [], [], []
    for lvl in levels:
        C = _IN_CHANNELS[lvl]
        A = _NUM_ANCHORS[lvl]
        W = _FEAT_HW[lvl]
        HW = W * W
        L = HW + 2 * W + 2
        Cout = (_NUM_CLASSES + 4) * A
        x_specs.append(pl.BlockSpec((1, C, HW), lambda n: (n, 0, 0)))
        w_specs.append(pl.BlockSpec((9, C, Cout), lambda n: (0, 0, 0)))
        b_specs.append(pl.BlockSpec((1, Cout), lambda n: (0, 0)))
        cls_specs.append(pl.BlockSpec((1, HW * A, _NUM_CLASSES),
                                      lambda n: (n, 0, 0)))
        reg_specs.append(pl.BlockSpec((1, HW, 4 * A), lambda n: (n, 0, 0)))
        scratch_shapes.append(pltpu.VMEM((L, C), jnp.bfloat16))
    out_shapes = (
        [jax.ShapeDtypeStruct((N, _FEAT_HW[l] ** 2 * _NUM_ANCHORS[l], _NUM_CLASSES),
                              jnp.float32) for l in levels]
        + [jax.ShapeDtypeStruct((N, _FEAT_HW[l] ** 2, 4 * _NUM_ANCHORS[l]),
                                jnp.float32) for l in levels]
    )
    return pl.pallas_call(
        functools.partial(_head_kernel, levels=levels),
        grid=(N,),
        in_specs=x_specs + w_specs + b_specs,
        out_specs=cls_specs + reg_specs,
        out_shape=out_shapes,
        scratch_shapes=scratch_shapes,
        compiler_params=pltpu.CompilerParams(
            dimension_semantics=("arbitrary",),
        ),
    )(*[xrs[l] for l in levels], *[wcs[l] for l in levels],
      *[bcs[l] for l in levels])


def kernel(x0, x1, x2, x3, x4, x5, reg_w0, reg_w1, reg_w2, reg_w3, reg_w4, reg_w5, reg_b0, reg_b1, reg_b2, reg_b3, reg_b4, reg_b5, cls_w0, cls_w1, cls_w2, cls_w3, cls_w4, cls_w5, cls_b0, cls_b1, cls_b2, cls_b3, cls_b4, cls_b5):
    xs = [x0, x1, x2, x3, x4, x5]
    reg_ws = [reg_w0, reg_w1, reg_w2, reg_w3, reg_w4, reg_w5]
    reg_bs = [reg_b0, reg_b1, reg_b2, reg_b3, reg_b4, reg_b5]
    cls_ws = [cls_w0, cls_w1, cls_w2, cls_w3, cls_w4, cls_w5]
    cls_bs = [cls_b0, cls_b1, cls_b2, cls_b3, cls_b4, cls_b5]
    N = x0.shape[0]

    xrs, wcs, bcs = [], [], []
    for i in range(6):
        C = _IN_CHANNELS[i]
        HW = _FEAT_HW[i] ** 2
        Cout = (_NUM_CLASSES + 4) * _NUM_ANCHORS[i]
        xrs.append(xs[i].reshape(N, C, HW))
        wc = jnp.concatenate([cls_ws[i], reg_ws[i]], axis=0)
        wc = wc.astype(jnp.bfloat16)
        wcs.append(jnp.transpose(wc, (2, 3, 1, 0)).reshape(9, C, Cout))
        bcs.append(jnp.concatenate([cls_bs[i], reg_bs[i]])[None, :])

    cls_parts = [None] * 6
    reg_parts = [None] * 6
    for levels in _LEVEL_GROUPS:
        outs = _group_call(levels, xrs, wcs, bcs, N)
        nl = len(levels)
        for j, lvl in enumerate(levels):
            HWA = _FEAT_HW[lvl] ** 2 * _NUM_ANCHORS[lvl]
            cls_parts[lvl] = outs[j]
            reg_parts[lvl] = outs[nl + j].reshape(N, HWA, 4)

    bbox_regression = jnp.concatenate(reg_parts, axis=1)
    cls_logits = jnp.concatenate(cls_parts, axis=1)
    return (bbox_regression, cls_logits)


# speedup vs baseline: 1.1278x; 1.0089x over previous
import functools

import jax
import jax.numpy as jnp
from jax.experimental import pallas as pl
from jax.experimental.pallas import tpu as pltpu

_IN_CHANNELS = [512, 1024, 512, 256, 256, 256]
_NUM_ANCHORS = [4, 6, 6, 6, 4, 4]
_NUM_CLASSES = 91
_FEAT_HW = [64, 32, 16, 8, 4, 2]
_LEVEL_GROUPS = ((0,), (1, 2, 3, 4, 5))
_TOTAL_ROWS = sum(_FEAT_HW[i] ** 2 * _NUM_ANCHORS[i] for i in range(6))
_ROW_STARTS = [0]
for _i in range(6):
    _ROW_STARTS.append(_ROW_STARTS[-1] + _FEAT_HW[_i] ** 2 * _NUM_ANCHORS[_i])


def _head_kernel(*refs, levels, cls_base):
    nl = len(levels)
    x_refs = refs[0:nl]
    w_refs = refs[nl:2 * nl]
    b_refs = refs[2 * nl:3 * nl]
    carry = 1 if cls_base > 0 else 0
    cls_ref = refs[3 * nl + carry]
    reg_refs = refs[3 * nl + carry + 1:3 * nl + carry + 1 + nl]
    xp_refs = refs[3 * nl + carry + 1 + nl:]

    first_step = pl.program_id(0) == 0
    for i, lvl in enumerate(levels):
        H = W = _FEAT_HW[lvl]
        A = _NUM_ANCHORS[lvl]
        C = _IN_CHANNELS[lvl]
        HW = H * W
        P = W + 1
        L = HW + 2 * W + 2
        ncls = _NUM_CLASSES * A
        x_ref, w_ref, b_ref = x_refs[i], w_refs[i], b_refs[i]
        reg_ref, xp_ref = reg_refs[i], xp_refs[i]
        local_start = _ROW_STARTS[lvl] - cls_base

        @pl.when(first_step)
        def _zero_borders(xp_ref=xp_ref, P=P, HW=HW, L=L, C=C):
            xp_ref[pl.ds(0, P), :] = jnp.zeros((P, C), jnp.bfloat16)
            xp_ref[pl.ds(P + HW, L - P - HW), :] = jnp.zeros(
                (L - P - HW, C), jnp.bfloat16)

        xp_ref[pl.ds(P, HW), :] = jnp.transpose(
            x_ref[0].astype(jnp.bfloat16), (1, 0))

        total = None
        for dx in (-1, 0, 1):
            acc = None
            for dy in (-1, 0, 1):
                t = (dy + 1) * 3 + (dx + 1)
                xs = xp_ref[pl.ds(P + dy * W + dx, HW), :]
                m = jnp.dot(xs, w_ref[t], preferred_element_type=jnp.float32)
                acc = m if acc is None else acc + m
            if dx != 0:
                col = jax.lax.broadcasted_iota(jnp.int32, (HW, 1), 0) % W
                bad = col == (0 if dx == -1 else W - 1)
                acc = jnp.where(bad, 0.0, acc)
            total = acc if total is None else total + acc
        total = total + b_ref[...]
        for a in range(A):
            cls_ref[0, pl.Slice(local_start + a, HW, A), :] = (
                total[:, a * _NUM_CLASSES:(a + 1) * _NUM_CLASSES])
        reg_ref[0] = total[:, ncls:]


def _group_call(levels, xrs, wcs, bcs, N, cls_carry):
    x_specs, w_specs, b_specs = [], [], []
    reg_specs, scratch_shapes = [], []
    for lvl in levels:
        C = _IN_CHANNELS[lvl]
        A = _NUM_ANCHORS[lvl]
        W = _FEAT_HW[lvl]
        HW = W * W
        L = HW + 2 * W + 2
        Cout = (_NUM_CLASSES + 4) * A
        x_specs.append(pl.BlockSpec((1, C, HW), lambda n: (n, 0, 0)))
        w_specs.append(pl.BlockSpec((9, C, Cout), lambda n: (0, 0, 0)))
        b_specs.append(pl.BlockSpec((1, Cout), lambda n: (0, 0)))
        reg_specs.append(pl.BlockSpec((1, HW, 4 * A), lambda n: (n, 0, 0)))
        scratch_shapes.append(pltpu.VMEM((L, C), jnp.bfloat16))

    cls_base = _ROW_STARTS[levels[0]]
    if cls_carry is None:
        cls_block_rows = _ROW_STARTS[levels[-1] + 1]
        cls_spec = pl.BlockSpec((1, cls_block_rows, _NUM_CLASSES),
                                lambda n: (n, 0, 0))
        extra_in, extra_in_specs, aliases = (), [], {}
    else:
        cls_block_rows = 8192
        blk = cls_base // cls_block_rows
        cls_spec = pl.BlockSpec((1, cls_block_rows, _NUM_CLASSES),
                                lambda n, blk=blk: (n, blk, 0))
        extra_in = (cls_carry,)
        extra_in_specs = [pl.BlockSpec(memory_space=pl.ANY)]
        aliases = {3 * len(levels): 0}

    out_shapes = (
        [jax.ShapeDtypeStruct((N, _TOTAL_ROWS, _NUM_CLASSES), jnp.float32)]
        + [jax.ShapeDtypeStruct((N, _FEAT_HW[l] ** 2, 4 * _NUM_ANCHORS[l]),
                                jnp.float32) for l in levels]
    )
    outs = pl.pallas_call(
        functools.partial(_head_kernel, levels=levels, cls_base=cls_base),
        grid=(N,),
        in_specs=x_specs + w_specs + b_specs + extra_in_specs,
        out_specs=[cls_spec] + reg_specs,
        out_shape=out_shapes,
        scratch_shapes=scratch_shapes,
        input_output_aliases=aliases,
        compiler_params=pltpu.CompilerParams(
            dimension_semantics=("arbitrary",),
        ),
    )(*[xrs[l] for l in levels], *[wcs[l] for l in levels],
      *[bcs[l] for l in levels], *extra_in)
    return outs[0], outs[1:]


def kernel(x0, x1, x2, x3, x4, x5, reg_w0, reg_w1, reg_w2, reg_w3, reg_w4, reg_w5, reg_b0, reg_b1, reg_b2, reg_b3, reg_b4, reg_b5, cls_w0, cls_w1, cls_w2, cls_w3, cls_w4, cls_w5, cls_b0, cls_b1, cls_b2, cls_b3, cls_b4, cls_b5):
    xs = [x0, x1, x2, x3, x4, x5]
    reg_ws = [reg_w0, reg_w1, reg_w2, reg_w3, reg_w4, reg_w5]
    reg_bs = [reg_b0, reg_b1, reg_b2, reg_b3, reg_b4, reg_b5]
    cls_ws = [cls_w0, cls_w1, cls_w2, cls_w3, cls_w4, cls_w5]
    cls_bs = [cls_b0, cls_b1, cls_b2, cls_b3, cls_b4, cls_b5]
    N = x0.shape[0]

    xrs, wcs, bcs = [], [], []
    for i in range(6):
        C = _IN_CHANNELS[i]
        Cout = (_NUM_CLASSES + 4) * _NUM_ANCHORS[i]
        xrs.append(xs[i].reshape(N, C, _FEAT_HW[i] ** 2))
        wc = jnp.concatenate([cls_ws[i], reg_ws[i]], axis=0)
        wc = wc.astype(jnp.bfloat16)
        wcs.append(jnp.transpose(wc, (2, 3, 1, 0)).reshape(9, C, Cout))
        bcs.append(jnp.concatenate([cls_bs[i], reg_bs[i]])[None, :])

    reg_parts = [None] * 6
    cls_logits = None
    for levels in _LEVEL_GROUPS:
        cls_logits, regs = _group_call(levels, xrs, wcs, bcs, N, cls_logits)
        for j, lvl in enumerate(levels):
            HWA = _FEAT_HW[lvl] ** 2 * _NUM_ANCHORS[lvl]
            reg_parts[lvl] = regs[j].reshape(N, HWA, 4)

    bbox_regression = jnp.concatenate(reg_parts, axis=1)
    return (bbox_regression, cls_logits)
